# baseline (speedup 1.0000x reference)
import functools

import jax
import jax.numpy as jnp
from jax import lax
from jax.experimental import pallas as pl
from jax.experimental.pallas import tpu as pltpu
from jax.experimental.pallas import tpu_sc as plsc

_LANES = 128


def _table_body(w_ref, b_ref, out_ref):
    out_ref[...] = w_ref[...].T + b_ref[...]


def _build_table(w_pad, b_row):
    e, cp = w_pad.shape
    return pl.pallas_call(
        _table_body,
        out_shape=jax.ShapeDtypeStruct((cp, e), w_pad.dtype),
    )(w_pad, b_row)


def _gather_rows(table, idx3, n_i, t):
    cp, e = table.shape
    nw, k, chunk = idx3.shape
    ipc = chunk // t
    i_per_tile = n_i // nw
    group = 2
    nbuf = 2 * group
    mesh = plsc.VectorSubcoreMesh(core_axis_name="c", subcore_axis_name="s")

    @functools.partial(
        pl.kernel,
        out_type=jax.ShapeDtypeStruct((n_i, t, e), jnp.float32),
        mesh=mesh,
        scratch_types=[
            pltpu.VMEM_SHARED((cp, e), jnp.float32),
            pltpu.VMEM((k, chunk), jnp.int32),
            pltpu.VMEM((nbuf, chunk, e), jnp.float32),
            [pltpu.SemaphoreType.DMA] * nbuf,
            [pltpu.SemaphoreType.DMA] * nbuf,
        ],
    )
    def k_fn(table_hbm, idx_hbm, out_hbm, table_sh, idx_v, rows_v, sems_g, sems_s):
        sid = lax.axis_index("s")
        wid = sid * 2 + lax.axis_index("c")
        i_base = wid * i_per_tile

        @pl.when(sid == 0)
        def _stage():
            pltpu.sync_copy(table_hbm, table_sh)

        pltpu.sync_copy(idx_hbm.at[wid], idx_v)
        plsc.subcore_barrier()

        def drain_stores(p):
            for u in range(group):
                pltpu.make_async_copy(
                    rows_v.at[p * group + u].reshape(ipc, t, e),
                    out_hbm.at[pl.ds(i_base, ipc)],
                    sems_s[p * group + u],
                ).wait()

        def fire_gathers(g, p):
            c0 = g * 2 * group + p * group
            return [
                pltpu.async_copy(
                    table_sh.at[idx_v.at[c0 + u]],
                    rows_v.at[p * group + u],
                    sems_g[p * group + u],
                )
                for u in range(group)
            ]

        def fire_stores(g, p, gathers):
            c0 = g * 2 * group + p * group
            for u in range(group):
                gathers[u].wait()
                i0 = i_base + (c0 + u) * ipc
                pltpu.async_copy(
                    rows_v.at[p * group + u].reshape(ipc, t, e),
                    out_hbm.at[pl.ds(i0, ipc)],
                    sems_s[p * group + u],
                )

        def body(g, carry):
            @pl.when(g > 0)
            def _():
                drain_stores(0)

            ga = fire_gathers(g, 0)

            @pl.when(g > 0)
            def _():
                drain_stores(1)

            gb = fire_gathers(g, 1)
            fire_stores(g, 0, ga)
            fire_stores(g, 1, gb)
            return carry

        lax.fori_loop(0, k // (2 * group), body, 0)
        drain_stores(0)
        drain_stores(1)

    return k_fn(table, idx3)


def kernel(x, W, b):
    e, c = W.shape
    cp = (c + _LANES - 1) // _LANES * _LANES
    w_pad = jnp.pad(W, ((0, 0), (0, cp - c)))
    table = _build_table(w_pad, b.reshape(1, e))

    n_i, t = x.shape
    nw = 32
    ipc = 4
    chunk = ipc * t
    k = n_i // (nw * ipc)
    idx3 = x.reshape(nw, k, chunk).astype(jnp.int32)
    return _gather_rows(table, idx3, n_i, t)


# speedup vs baseline: 1.9064x; 1.9064x over previous
import functools

import jax
import jax.numpy as jnp
from jax import lax
from jax.experimental import pallas as pl
from jax.experimental.pallas import tpu as pltpu
from jax.experimental.pallas import tpu_sc as plsc


def _table_body(wt_ref, b_ref, out_ref):
    out_ref[...] = wt_ref[...] + b_ref[...]


def _build_table(wt, b_row):
    return pl.pallas_call(
        _table_body,
        out_shape=jax.ShapeDtypeStruct(wt.shape, wt.dtype),
    )(wt, b_row)


def _gather_rows(table, idx3):
    c, e = table.shape
    nw, k, chunk = idx3.shape
    rows_per_w = k * chunk
    group = 2
    nbuf = 2 * group
    mesh = plsc.VectorSubcoreMesh(core_axis_name="c", subcore_axis_name="s")

    @functools.partial(
        pl.kernel,
        out_type=jax.ShapeDtypeStruct((nw * rows_per_w, e), jnp.float32),
        mesh=mesh,
        scratch_types=[
            pltpu.VMEM_SHARED((c, e), jnp.float32),
            pltpu.VMEM((k, chunk), jnp.int32),
            pltpu.VMEM((nbuf, chunk, e), jnp.float32),
            [pltpu.SemaphoreType.DMA] * nbuf,
            [pltpu.SemaphoreType.DMA] * nbuf,
        ],
    )
    def k_fn(table_hbm, idx_hbm, out_hbm, table_sh, idx_v, rows_v, sems_g, sems_s):
        sid = lax.axis_index("s")
        wid = sid * 2 + lax.axis_index("c")
        base = wid * rows_per_w

        @pl.when(sid == 0)
        def _stage():
            pltpu.sync_copy(table_hbm, table_sh)

        pltpu.sync_copy(idx_hbm.at[wid], idx_v)
        plsc.subcore_barrier()

        def drain_stores(p):
            for u in range(group):
                pltpu.make_async_copy(
                    rows_v.at[p * group + u],
                    out_hbm.at[pl.ds(base, chunk)],
                    sems_s[p * group + u],
                ).wait()

        def fire_gathers(g, p):
            c0 = g * 2 * group + p * group
            return [
                pltpu.async_copy(
                    table_sh.at[idx_v.at[c0 + u]],
                    rows_v.at[p * group + u],
                    sems_g[p * group + u],
                )
                for u in range(group)
            ]

        def fire_stores(g, p, gathers):
            c0 = g * 2 * group + p * group
            for u in range(group):
                gathers[u].wait()
                pltpu.async_copy(
                    rows_v.at[p * group + u],
                    out_hbm.at[pl.ds(base + (c0 + u) * chunk, chunk)],
                    sems_s[p * group + u],
                )

        def body(g, carry):
            @pl.when(g > 0)
            def _():
                drain_stores(0)

            ga = fire_gathers(g, 0)

            @pl.when(g > 0)
            def _():
                drain_stores(1)

            gb = fire_gathers(g, 1)
            fire_stores(g, 0, ga)
            fire_stores(g, 1, gb)
            return carry

        lax.fori_loop(0, k // (2 * group), body, 0)
        drain_stores(0)
        drain_stores(1)

    return k_fn(table, idx3)


def kernel(x, W, b):
    e, c = W.shape
    n_i, t = x.shape
    table = _build_table(W.T, b.reshape(1, e))

    nw = 32
    chunk = 128
    k = x.size // (nw * chunk)
    idx3 = x.T.reshape(nw, k, chunk).astype(jnp.int32)
    out = _gather_rows(table, idx3)
    return out.reshape(t, n_i, e).transpose(1, 0, 2)


# speedup vs baseline: 1.9102x; 1.0020x over previous
import functools

import jax
import jax.numpy as jnp
from jax import lax
from jax.experimental import pallas as pl
from jax.experimental.pallas import tpu as pltpu
from jax.experimental.pallas import tpu_sc as plsc


def _table_body(wt_ref, b_ref, out_ref):
    out_ref[...] = wt_ref[...] + b_ref[...]


def _build_table(wt, b_row):
    return pl.pallas_call(
        _table_body,
        out_shape=jax.ShapeDtypeStruct(wt.shape, wt.dtype),
    )(wt, b_row)


def _gather_rows(table, idx3):
    c, e = table.shape
    (n,) = idx3.shape
    nw = 32
    chunk = 128
    rows_per_w = n // nw
    k = rows_per_w // chunk
    group = 2
    nbuf = 2 * group
    mesh = plsc.VectorSubcoreMesh(core_axis_name="c", subcore_axis_name="s")

    @functools.partial(
        pl.kernel,
        out_type=jax.ShapeDtypeStruct((nw * rows_per_w, e), jnp.float32),
        mesh=mesh,
        scratch_types=[
            pltpu.VMEM_SHARED((c, e), jnp.float32),
            pltpu.VMEM((rows_per_w,), jnp.int32),
            pltpu.VMEM((nbuf, chunk, e), jnp.float32),
            [pltpu.SemaphoreType.DMA] * nbuf,
            [pltpu.SemaphoreType.DMA] * nbuf,
        ],
    )
    def k_fn(table_hbm, idx_hbm, out_hbm, table_sh, idx_v, rows_v, sems_g, sems_s):
        sid = lax.axis_index("s")
        wid = sid * 2 + lax.axis_index("c")
        base = wid * rows_per_w

        @pl.when(sid == 0)
        def _stage():
            pltpu.sync_copy(table_hbm, table_sh)

        pltpu.sync_copy(idx_hbm.at[pl.ds(base, rows_per_w)], idx_v)
        plsc.subcore_barrier()

        def drain_stores(p):
            for u in range(group):
                pltpu.make_async_copy(
                    rows_v.at[p * group + u],
                    out_hbm.at[pl.ds(base, chunk)],
                    sems_s[p * group + u],
                ).wait()

        def fire_gathers(g, p):
            c0 = g * 2 * group + p * group
            return [
                pltpu.async_copy(
                    table_sh.at[idx_v.at[pl.ds((c0 + u) * chunk, chunk)]],
                    rows_v.at[p * group + u],
                    sems_g[p * group + u],
                )
                for u in range(group)
            ]

        def fire_stores(g, p, gathers):
            c0 = g * 2 * group + p * group
            for u in range(group):
                gathers[u].wait()
                pltpu.async_copy(
                    rows_v.at[p * group + u],
                    out_hbm.at[pl.ds(base + (c0 + u) * chunk, chunk)],
                    sems_s[p * group + u],
                )

        def body(g, carry):
            @pl.when(g > 0)
            def _():
                drain_stores(0)

            ga = fire_gathers(g, 0)

            @pl.when(g > 0)
            def _():
                drain_stores(1)

            gb = fire_gathers(g, 1)
            fire_stores(g, 0, ga)
            fire_stores(g, 1, gb)
            return carry

        lax.fori_loop(0, k // (2 * group), body, 0)
        drain_stores(0)
        drain_stores(1)

    return k_fn(table, idx3)


def kernel(x, W, b):
    e, c = W.shape
    n_i, t = x.shape
    table = _build_table(W.T, b.reshape(1, e))

    idx = x.T.reshape(x.size).astype(jnp.int32)
    out = _gather_rows(table, idx)
    return out.reshape(t, n_i, e).transpose(1, 0, 2)
